# baseline (speedup 1.0000x reference)
import functools

import jax
import jax.numpy as jnp
from jax import lax
from jax.experimental import pallas as pl
from jax.experimental.pallas import tpu as pltpu
from jax.experimental.pallas import tpu_sc as plsc

N = 50000
D = 256
S = 256
L = 16
NC = 2
NS = 16
NW = NC * NS
BR = 112
NBLK = (N + BR - 1) // BR
KMAX = (NBLK + NW - 1) // NW
H = D // 2
FH = H // L
FV = D // L

_f32 = jnp.float32
_i32 = jnp.int32


RPT = 1568
SBR = 32
SKB = RPT // SBR


def _stats_body(x_hbm, batch_hbm, psum, psq, pcnt,
                idxall, xb0, xb1, asum, asq, acnt, xs0, xs1):
    cid = lax.axis_index("c")
    sid = lax.axis_index("s")
    wid = sid * NC + cid
    xb = (xb0, xb1)
    xsem = (xs0, xs1)
    ones = jnp.ones((L,), _f32)
    sixteens = jnp.full((L,), 16.0, _f32)
    iota = lax.iota(_i32, L)

    base = RPT * wid
    tend = jnp.minimum(base + RPT, N)
    astart = pl.multiple_of(jnp.minimum(base, N - RPT), 8)
    pltpu.sync_copy(batch_hbm.at[pl.ds(astart, RPT)], idxall)
    nblk = (tend - base + SBR - 1) // SBR

    def pstart(k):
        return pl.multiple_of(jnp.minimum(base + SBR * k, N - SBR), 8)

    for h in range(2):
        def zero_body(r, _):
            for f in range(FH):
                asum[r, pl.ds(f * L, L)] = jnp.zeros((L,), _f32)
                asq[r, pl.ds(f * L, L)] = jnp.zeros((L,), _f32)
            if h == 0:
                acnt[r, pl.ds(0, L)] = jnp.zeros((L,), _f32)
            return 0
        lax.fori_loop(0, S + 1, zero_body, 0)

        def issue(k, i):
            pltpu.async_copy(
                x_hbm.at[pl.ds(pstart(k), SBR), pl.ds(h * H, H)],
                xb[i], xsem[i])

        for i in range(2):
            @pl.when(i < nblk)
            def _():
                issue(i, i)

        def ring_body(t, _):
            for i in range(2):
                k = 2 * t + i

                @pl.when(k < nblk)
                def _():
                    ps = pstart(k)
                    pltpu.make_async_copy(
                        x_hbm.at[pl.ds(ps, SBR), pl.ds(h * H, H)],
                        xb[i], xsem[i]).wait()
                    vfrom = base + SBR * k
                    vto = jnp.minimum(vfrom + SBR, tend)
                    loffk = ps - astart

                    def group_body(g, _):
                        bv = idxall[pl.ds(loffk + g * L, L)]
                        pos = ps + g * L + iota
                        validv = (pos >= vfrom) & (pos < vto)
                        u = jnp.where(validv, bv, S)
                        u0 = u[0]
                        uniform = (u0 == u[L - 1]) & (u0 < S)

                        @pl.when(uniform)
                        def _():
                            seg = u0
                            for f in range(FH):
                                acs = jnp.zeros((L,), _f32)
                                acq = jnp.zeros((L,), _f32)
                                for j in range(L):
                                    xv = xb[i][g * L + j, pl.ds(f * L, L)]
                                    acs = acs + xv
                                    acq = acq + xv * xv
                                plsc.addupdate(
                                    asum.at[seg, pl.ds(f * L, L)], acs)
                                plsc.addupdate(
                                    asq.at[seg, pl.ds(f * L, L)], acq)
                            if h == 0:
                                plsc.addupdate(
                                    acnt.at[seg, pl.ds(0, L)], sixteens)

                        @pl.when(jnp.logical_not(uniform))
                        def _():
                            for j in range(L):
                                seg = u[j]
                                for f in range(FH):
                                    xv = xb[i][g * L + j, pl.ds(f * L, L)]
                                    plsc.addupdate(
                                        asum.at[seg, pl.ds(f * L, L)], xv)
                                    plsc.addupdate(
                                        asq.at[seg, pl.ds(f * L, L)],
                                        xv * xv)
                                if h == 0:
                                    plsc.addupdate(
                                        acnt.at[seg, pl.ds(0, L)], ones)
                        return 0
                    lax.fori_loop(0, SBR // L, group_body, 0)

                    @pl.when(k + 2 < nblk)
                    def _():
                        issue(k + 2, i)
            return 0

        lax.fori_loop(0, (SKB + 1) // 2, ring_body, 0)

        pltpu.sync_copy(asum, psum.at[h, wid])
        pltpu.sync_copy(asq, psq.at[h, wid])
        if h == 0:
            pltpu.sync_copy(acnt, pcnt.at[wid])


@functools.cache
def _make_stats():
  mesh = plsc.VectorSubcoreMesh(
      core_axis_name="c", subcore_axis_name="s",
      num_cores=NC, num_subcores=NS)
  return functools.partial(
    pl.kernel,
    mesh=mesh,
    out_type=(
        jax.ShapeDtypeStruct((2, NW, S + 1, H), _f32),
        jax.ShapeDtypeStruct((2, NW, S + 1, H), _f32),
        jax.ShapeDtypeStruct((NW, S + 1, L), _f32),
    ),
    scratch_types=[
        pltpu.VMEM((RPT,), _i32),
        pltpu.VMEM((SBR, H), _f32),
        pltpu.VMEM((SBR, H), _f32),
        pltpu.VMEM((S + 1, H), _f32),
        pltpu.VMEM((S + 1, H), _f32),
        pltpu.VMEM((S + 1, L), _f32),
        pltpu.SemaphoreType.DMA,
        pltpu.SemaphoreType.DMA,
    ],
  )(_stats_body)


def _finalize_body(psum_ref, psq_ref, pcnt_ref, scale_ref, bias_ref, ab_ref):
    s0 = jnp.sum(psum_ref[0], axis=0)[:S]
    s1 = jnp.sum(psum_ref[1], axis=0)[:S]
    q0 = jnp.sum(psq_ref[0], axis=0)[:S]
    q1 = jnp.sum(psq_ref[1], axis=0)[:S]
    s = jnp.concatenate([s0, s1], axis=1)
    q = jnp.concatenate([q0, q1], axis=1)
    c = jnp.sum(pcnt_ref[...], axis=0)[:S, 0:1]
    c_safe = jnp.maximum(c, 1.0)
    mean = s / c_safe
    denom = jnp.maximum(c - 1.0, 1.0)
    var = jnp.maximum((q - c * mean * mean) / denom, 0.0)
    std = jnp.sqrt(var)
    a = scale_ref[...][None, :] / (std + 1e-5)
    bb = bias_ref[...][None, :] - mean * a
    ab_ref[0:S, 0:D] = a
    ab_ref[0:S, D:2 * D] = bb
    ab_ref[S:, :] = jnp.zeros((16, 2 * D), _f32)


def _finalize(psum, psq, pcnt, scale, bias):
    return pl.pallas_call(
        _finalize_body,
        out_shape=jax.ShapeDtypeStruct((S + 16, 2 * D), _f32),
    )(psum, psq, pcnt, scale, bias)


NRPT = 1600
NB = 96
NKB = (NRPT + NB - 1) // NB
W = 8


def _norm_body(x_hbm, batch_hbm, ab_hbm, out_hbm,
               ib0, ib1, xb0, xb1, ob0, ob1, aw0, aw1, tmp,
               is0, is1, xs0, xs1, as0, as1, os0, os1):
    cid = lax.axis_index("c")
    sid = lax.axis_index("s")
    wid = sid * NC + cid
    ib = (ib0, ib1)
    xb = (xb0, xb1)
    ob = (ob0, ob1)
    aw = (aw0, aw1)
    isem = (is0, is1)
    xsem = (xs0, xs1)
    asem = (as0, as1)
    osem = (os0, os1)

    base = NRPT * wid
    tend = jnp.minimum(base + NRPT, N)
    nblk = (tend - base + NB - 1) // NB

    def pstart(k):
        return pl.multiple_of(jnp.minimum(base + NB * k, N - NB), 8)

    def u0last(i):
        v0 = ib[i][pl.ds(0, L)]
        v2 = ib[i][pl.ds(NB - L, L)]
        return v0[0], v2[L - 1]

    def issue_ix(k, i):
        ps = pstart(k)
        pltpu.async_copy(batch_hbm.at[pl.ds(ps, NB)], ib[i], isem[i])
        pltpu.async_copy(x_hbm.at[pl.ds(ps, NB)], xb[i], xsem[i])

    def issue_ab(i):
        u0, ul = u0last(i)

        @pl.when(ul - u0 < W)
        def _():
            off = pl.multiple_of(u0 * 2 * D, 8)
            pltpu.async_copy(ab_hbm.at[pl.ds(off, W * 2 * D)], aw[i], asem[i])

    def wait_ab(i):
        u0, ul = u0last(i)

        @pl.when(ul - u0 < W)
        def _():
            off = pl.multiple_of(u0 * 2 * D, 8)
            pltpu.make_async_copy(
                ab_hbm.at[pl.ds(off, W * 2 * D)], aw[i], asem[i]).wait()

    issue_ix(0, 0)
    pltpu.make_async_copy(
        batch_hbm.at[pl.ds(pstart(0), NB)], ib[0], isem[0]).wait()
    issue_ab(0)

    @pl.when(1 < nblk)
    def _():
        issue_ix(1, 1)

    def pair_body(t, _):
        for i in range(2):
            k = 2 * t + i
            inext = (i + 1) & 1

            @pl.when(k < nblk)
            def _():
                ps = pstart(k)

                @pl.when(k + 1 < nblk)
                def _():
                    pltpu.make_async_copy(
                        batch_hbm.at[pl.ds(pstart(k + 1), NB)],
                        ib[inext], isem[inext]).wait()
                    issue_ab(inext)

                pltpu.make_async_copy(
                    x_hbm.at[pl.ds(ps, NB)], xb[i], xsem[i]).wait()
                wait_ab(i)

                @pl.when(k >= 2)
                def _():
                    pltpu.make_async_copy(
                        ob[i], out_hbm.at[pl.ds(pstart(k - 2), NB)],
                        osem[i]).wait()

                u0, ul = u0last(i)

                @pl.when(ul - u0 < W)
                def _():
                    def group_body(g, _):
                        idv = ib[i][pl.ds(g * L, L)]
                        guni = idv[0] == idv[L - 1]

                        @pl.when(guni)
                        def _():
                            rbase = (idv[0] - u0) * 2 * D

                            def cbody(c, _):
                                for ff in range(4):
                                    off = c * 4 * L + ff * L
                                    av = aw[i][pl.ds(rbase + off, L)]
                                    bv = aw[i][pl.ds(rbase + D + off, L)]
                                    for j in range(L):
                                        r = g * L + j
                                        ob[i][r, pl.ds(off, L)] = (
                                            xb[i][r, pl.ds(off, L)] * av
                                            + bv)
                                return 0
                            lax.fori_loop(0, FV // 4, cbody, 0)

                        @pl.when(jnp.logical_not(guni))
                        def _():
                            for j in range(L):
                                r = g * L + j
                                rbase = (idv[j] - u0) * 2 * D

                                def fbody(f, _):
                                    off = f * L
                                    av = aw[i][pl.ds(rbase + off, L)]
                                    bv = aw[i][pl.ds(rbase + D + off, L)]
                                    ob[i][r, pl.ds(off, L)] = (
                                        xb[i][r, pl.ds(off, L)] * av + bv)
                                    return 0
                                lax.fori_loop(0, FV, fbody, 0)
                        return 0
                    lax.fori_loop(0, NB // L, group_body, 0)

                @pl.when(ul - u0 >= W)
                def _():
                    def sgroup_body(g, _):
                        idv = ib[i][pl.ds(g * L, L)]
                        for j in range(L):
                            r = g * L + j
                            soff = pl.multiple_of(idv[j] * 2 * D, 8)
                            pltpu.sync_copy(
                                ab_hbm.at[pl.ds(soff, 2 * D)], tmp)

                            def fbody(f, _):
                                off = f * L
                                av = tmp[pl.ds(off, L)]
                                bv = tmp[pl.ds(D + off, L)]
                                ob[i][r, pl.ds(off, L)] = (
                                    xb[i][r, pl.ds(off, L)] * av + bv)
                                return 0
                            lax.fori_loop(0, FV, fbody, 0)
                        return 0
                    lax.fori_loop(0, NB // L, sgroup_body, 0)

                pltpu.async_copy(ob[i], out_hbm.at[pl.ds(ps, NB)], osem[i])

                @pl.when(k + 2 < nblk)
                def _():
                    issue_ix(k + 2, i)
        return 0

    lax.fori_loop(0, (NKB + 1) // 2, pair_body, 0)

    for i in range(2):
        @pl.when(i < nblk)
        def _():
            pltpu.make_async_copy(
                ob[i], out_hbm.at[pl.ds(pstart(0), NB)], osem[i]).wait()


@functools.cache
def _make_norm():
  mesh = plsc.VectorSubcoreMesh(
      core_axis_name="c", subcore_axis_name="s",
      num_cores=NC, num_subcores=NS)
  return functools.partial(
    pl.kernel,
    mesh=mesh,
    out_type=jax.ShapeDtypeStruct((N, D), _f32),
    scratch_types=[
        pltpu.VMEM((NB,), _i32),
        pltpu.VMEM((NB,), _i32),
        pltpu.VMEM((NB, D), _f32),
        pltpu.VMEM((NB, D), _f32),
        pltpu.VMEM((NB, D), _f32),
        pltpu.VMEM((NB, D), _f32),
        pltpu.VMEM((W * 2 * D,), _f32),
        pltpu.VMEM((W * 2 * D,), _f32),
        pltpu.VMEM((2 * D,), _f32),
        pltpu.SemaphoreType.DMA,
        pltpu.SemaphoreType.DMA,
        pltpu.SemaphoreType.DMA,
        pltpu.SemaphoreType.DMA,
        pltpu.SemaphoreType.DMA,
        pltpu.SemaphoreType.DMA,
        pltpu.SemaphoreType.DMA,
        pltpu.SemaphoreType.DMA,
    ],
  )(_norm_body)


def kernel(x, batch, scale, bias):
    psum, psq, pcnt = _make_stats()(x, batch)
    ab = _finalize(psum, psq, pcnt, scale, bias)
    return _make_norm()(x, batch, ab.reshape(-1))


# speedup vs baseline: 1.0080x; 1.0080x over previous
import functools

import jax
import jax.numpy as jnp
from jax import lax
from jax.experimental import pallas as pl
from jax.experimental.pallas import tpu as pltpu
from jax.experimental.pallas import tpu_sc as plsc

N = 50000
D = 256
S = 256
L = 16
NC = 2
NS = 16
NW = NC * NS
BR = 112
NBLK = (N + BR - 1) // BR
KMAX = (NBLK + NW - 1) // NW
H = D // 2
FH = H // L
FV = D // L

_f32 = jnp.float32
_i32 = jnp.int32


RPT = 1568
SBR = 32
SKB = RPT // SBR


def _stats_body(x_hbm, batch_hbm, psum, psq, pcnt,
                idxall, xb0, xb1, asum, asq, acnt, xs0, xs1):
    cid = lax.axis_index("c")
    sid = lax.axis_index("s")
    wid = sid * NC + cid
    xb = (xb0, xb1)
    xsem = (xs0, xs1)
    ones = jnp.ones((L,), _f32)
    sixteens = jnp.full((L,), 16.0, _f32)
    iota = lax.iota(_i32, L)

    base = RPT * wid
    tend = jnp.minimum(base + RPT, N)
    astart = pl.multiple_of(jnp.minimum(base, N - RPT), 8)
    pltpu.sync_copy(batch_hbm.at[pl.ds(astart, RPT)], idxall)
    nblk = (tend - base + SBR - 1) // SBR

    def pstart(k):
        return pl.multiple_of(jnp.minimum(base + SBR * k, N - SBR), 8)

    for h in range(2):
        def zero_body(r, _):
            for f in range(FH):
                asum[r, pl.ds(f * L, L)] = jnp.zeros((L,), _f32)
                asq[r, pl.ds(f * L, L)] = jnp.zeros((L,), _f32)
            if h == 0:
                acnt[r, pl.ds(0, L)] = jnp.zeros((L,), _f32)
            return 0
        lax.fori_loop(0, S + 1, zero_body, 0)

        def issue(k, i):
            pltpu.async_copy(
                x_hbm.at[pl.ds(pstart(k), SBR), pl.ds(h * H, H)],
                xb[i], xsem[i])

        for i in range(2):
            @pl.when(i < nblk)
            def _():
                issue(i, i)

        def ring_body(t, _):
            for i in range(2):
                k = 2 * t + i

                @pl.when(k < nblk)
                def _():
                    ps = pstart(k)
                    pltpu.make_async_copy(
                        x_hbm.at[pl.ds(ps, SBR), pl.ds(h * H, H)],
                        xb[i], xsem[i]).wait()
                    vfrom = base + SBR * k
                    vto = jnp.minimum(vfrom + SBR, tend)
                    loffk = ps - astart

                    def group_body(g, _):
                        bv = idxall[pl.ds(loffk + g * L, L)]
                        pos = ps + g * L + iota
                        validv = (pos >= vfrom) & (pos < vto)
                        u = jnp.where(validv, bv, S)
                        u0 = u[0]
                        uniform = (u0 == u[L - 1]) & (u0 < S)

                        @pl.when(uniform)
                        def _():
                            seg = u0
                            for f in range(FH):
                                acs = jnp.zeros((L,), _f32)
                                acq = jnp.zeros((L,), _f32)
                                for j in range(L):
                                    xv = xb[i][g * L + j, pl.ds(f * L, L)]
                                    acs = acs + xv
                                    acq = acq + xv * xv
                                plsc.addupdate(
                                    asum.at[seg, pl.ds(f * L, L)], acs)
                                plsc.addupdate(
                                    asq.at[seg, pl.ds(f * L, L)], acq)
                            if h == 0:
                                plsc.addupdate(
                                    acnt.at[seg, pl.ds(0, L)], sixteens)

                        @pl.when(jnp.logical_not(uniform))
                        def _():
                            for j in range(L):
                                seg = u[j]
                                for f in range(FH):
                                    xv = xb[i][g * L + j, pl.ds(f * L, L)]
                                    plsc.addupdate(
                                        asum.at[seg, pl.ds(f * L, L)], xv)
                                    plsc.addupdate(
                                        asq.at[seg, pl.ds(f * L, L)],
                                        xv * xv)
                                if h == 0:
                                    plsc.addupdate(
                                        acnt.at[seg, pl.ds(0, L)], ones)
                        return 0
                    lax.fori_loop(0, SBR // L, group_body, 0)

                    @pl.when(k + 2 < nblk)
                    def _():
                        issue(k + 2, i)
            return 0

        lax.fori_loop(0, (SKB + 1) // 2, ring_body, 0)

        pltpu.sync_copy(asum, psum.at[h, wid])
        pltpu.sync_copy(asq, psq.at[h, wid])
        if h == 0:
            pltpu.sync_copy(acnt, pcnt.at[wid])


@functools.cache
def _make_stats():
  mesh = plsc.VectorSubcoreMesh(
      core_axis_name="c", subcore_axis_name="s",
      num_cores=NC, num_subcores=NS)
  return functools.partial(
    pl.kernel,
    mesh=mesh,
    out_type=(
        jax.ShapeDtypeStruct((2, NW, S + 1, H), _f32),
        jax.ShapeDtypeStruct((2, NW, S + 1, H), _f32),
        jax.ShapeDtypeStruct((NW, S + 1, L), _f32),
    ),
    scratch_types=[
        pltpu.VMEM((RPT,), _i32),
        pltpu.VMEM((SBR, H), _f32),
        pltpu.VMEM((SBR, H), _f32),
        pltpu.VMEM((S + 1, H), _f32),
        pltpu.VMEM((S + 1, H), _f32),
        pltpu.VMEM((S + 1, L), _f32),
        pltpu.SemaphoreType.DMA,
        pltpu.SemaphoreType.DMA,
    ],
  )(_stats_body)


def _finalize_body(psum_ref, psq_ref, pcnt_ref, scale_ref, bias_ref, ab_ref):
    s0 = jnp.sum(psum_ref[0], axis=0)[:S]
    s1 = jnp.sum(psum_ref[1], axis=0)[:S]
    q0 = jnp.sum(psq_ref[0], axis=0)[:S]
    q1 = jnp.sum(psq_ref[1], axis=0)[:S]
    s = jnp.concatenate([s0, s1], axis=1)
    q = jnp.concatenate([q0, q1], axis=1)
    c = jnp.sum(pcnt_ref[...], axis=0)[:S, 0:1]
    c_safe = jnp.maximum(c, 1.0)
    mean = s / c_safe
    denom = jnp.maximum(c - 1.0, 1.0)
    var = jnp.maximum((q - c * mean * mean) / denom, 0.0)
    std = jnp.sqrt(var)
    a = scale_ref[...][None, :] / (std + 1e-5)
    bb = bias_ref[...][None, :] - mean * a
    ab_ref[0:S, 0:D] = a
    ab_ref[0:S, D:2 * D] = bb
    ab_ref[S:, :] = jnp.zeros((16, 2 * D), _f32)


def _finalize(psum, psq, pcnt, scale, bias):
    return pl.pallas_call(
        _finalize_body,
        out_shape=jax.ShapeDtypeStruct((S + 16, 2 * D), _f32),
    )(psum, psq, pcnt, scale, bias)


NRPT = 1600
NB = 80
NKB = (NRPT + NB - 1) // NB
W = 8


def _norm_body(x_hbm, batch_hbm, ab_hbm, out_hbm,
               ib0, ib1, xb0, xb1, ob0, ob1, aw0, aw1, tmp,
               is0, is1, xs0, xs1, as0, as1, os0, os1):
    cid = lax.axis_index("c")
    sid = lax.axis_index("s")
    wid = sid * NC + cid
    ib = (ib0, ib1)
    xb = (xb0, xb1)
    ob = (ob0, ob1)
    aw = (aw0, aw1)
    isem = (is0, is1)
    xsem = (xs0, xs1)
    asem = (as0, as1)
    osem = (os0, os1)

    base = NRPT * wid
    tend = jnp.minimum(base + NRPT, N)
    nblk = (tend - base + NB - 1) // NB

    def pstart(k):
        return pl.multiple_of(jnp.minimum(base + NB * k, N - NB), 8)

    def u0last(i):
        v0 = ib[i][pl.ds(0, L)]
        v2 = ib[i][pl.ds(NB - L, L)]
        return v0[0], v2[L - 1]

    def issue_ix(k, i):
        ps = pstart(k)
        pltpu.async_copy(batch_hbm.at[pl.ds(ps, NB)], ib[i], isem[i])
        pltpu.async_copy(x_hbm.at[pl.ds(ps, NB)], xb[i], xsem[i])

    def issue_ab(i):
        u0, ul = u0last(i)

        @pl.when(ul - u0 < W)
        def _():
            off = pl.multiple_of(u0 * 2 * D, 8)
            pltpu.async_copy(ab_hbm.at[pl.ds(off, W * 2 * D)], aw[i], asem[i])

    def wait_ab(i):
        u0, ul = u0last(i)

        @pl.when(ul - u0 < W)
        def _():
            off = pl.multiple_of(u0 * 2 * D, 8)
            pltpu.make_async_copy(
                ab_hbm.at[pl.ds(off, W * 2 * D)], aw[i], asem[i]).wait()

    issue_ix(0, 0)
    pltpu.make_async_copy(
        batch_hbm.at[pl.ds(pstart(0), NB)], ib[0], isem[0]).wait()
    issue_ab(0)

    @pl.when(1 < nblk)
    def _():
        issue_ix(1, 1)

    def pair_body(t, _):
        for i in range(2):
            k = 2 * t + i
            inext = (i + 1) & 1

            @pl.when(k < nblk)
            def _():
                ps = pstart(k)

                @pl.when(k + 1 < nblk)
                def _():
                    pltpu.make_async_copy(
                        batch_hbm.at[pl.ds(pstart(k + 1), NB)],
                        ib[inext], isem[inext]).wait()
                    issue_ab(inext)

                pltpu.make_async_copy(
                    x_hbm.at[pl.ds(ps, NB)], xb[i], xsem[i]).wait()
                wait_ab(i)

                @pl.when(k >= 2)
                def _():
                    pltpu.make_async_copy(
                        ob[i], out_hbm.at[pl.ds(pstart(k - 2), NB)],
                        osem[i]).wait()

                u0, ul = u0last(i)

                @pl.when(ul - u0 < W)
                def _():
                    def group_body(g, _):
                        idv = ib[i][pl.ds(g * L, L)]
                        guni = idv[0] == idv[L - 1]

                        @pl.when(guni)
                        def _():
                            rbase = (idv[0] - u0) * 2 * D

                            def cbody(c, _):
                                for ff in range(4):
                                    off = c * 4 * L + ff * L
                                    av = aw[i][pl.ds(rbase + off, L)]
                                    bv = aw[i][pl.ds(rbase + D + off, L)]
                                    for j in range(L):
                                        r = g * L + j
                                        ob[i][r, pl.ds(off, L)] = (
                                            xb[i][r, pl.ds(off, L)] * av
                                            + bv)
                                return 0
                            lax.fori_loop(0, FV // 4, cbody, 0)

                        @pl.when(jnp.logical_not(guni))
                        def _():
                            for j in range(L):
                                r = g * L + j
                                rbase = (idv[j] - u0) * 2 * D

                                def fbody(f, _):
                                    off = f * L
                                    av = aw[i][pl.ds(rbase + off, L)]
                                    bv = aw[i][pl.ds(rbase + D + off, L)]
                                    ob[i][r, pl.ds(off, L)] = (
                                        xb[i][r, pl.ds(off, L)] * av + bv)
                                    return 0
                                lax.fori_loop(0, FV, fbody, 0)
                        return 0
                    lax.fori_loop(0, NB // L, group_body, 0)

                @pl.when(ul - u0 >= W)
                def _():
                    def sgroup_body(g, _):
                        idv = ib[i][pl.ds(g * L, L)]
                        for j in range(L):
                            r = g * L + j
                            soff = pl.multiple_of(idv[j] * 2 * D, 8)
                            pltpu.sync_copy(
                                ab_hbm.at[pl.ds(soff, 2 * D)], tmp)

                            def fbody(f, _):
                                off = f * L
                                av = tmp[pl.ds(off, L)]
                                bv = tmp[pl.ds(D + off, L)]
                                ob[i][r, pl.ds(off, L)] = (
                                    xb[i][r, pl.ds(off, L)] * av + bv)
                                return 0
                            lax.fori_loop(0, FV, fbody, 0)
                        return 0
                    lax.fori_loop(0, NB // L, sgroup_body, 0)

                pltpu.async_copy(ob[i], out_hbm.at[pl.ds(ps, NB)], osem[i])

                @pl.when(k + 2 < nblk)
                def _():
                    issue_ix(k + 2, i)
        return 0

    lax.fori_loop(0, (NKB + 1) // 2, pair_body, 0)

    for i in range(2):
        @pl.when(i < nblk)
        def _():
            pltpu.make_async_copy(
                ob[i], out_hbm.at[pl.ds(pstart(0), NB)], osem[i]).wait()


@functools.cache
def _make_norm():
  mesh = plsc.VectorSubcoreMesh(
      core_axis_name="c", subcore_axis_name="s",
      num_cores=NC, num_subcores=NS)
  return functools.partial(
    pl.kernel,
    mesh=mesh,
    out_type=jax.ShapeDtypeStruct((N, D), _f32),
    scratch_types=[
        pltpu.VMEM((NB,), _i32),
        pltpu.VMEM((NB,), _i32),
        pltpu.VMEM((NB, D), _f32),
        pltpu.VMEM((NB, D), _f32),
        pltpu.VMEM((NB, D), _f32),
        pltpu.VMEM((NB, D), _f32),
        pltpu.VMEM((W * 2 * D,), _f32),
        pltpu.VMEM((W * 2 * D,), _f32),
        pltpu.VMEM((2 * D,), _f32),
        pltpu.SemaphoreType.DMA,
        pltpu.SemaphoreType.DMA,
        pltpu.SemaphoreType.DMA,
        pltpu.SemaphoreType.DMA,
        pltpu.SemaphoreType.DMA,
        pltpu.SemaphoreType.DMA,
        pltpu.SemaphoreType.DMA,
        pltpu.SemaphoreType.DMA,
    ],
  )(_norm_body)


def kernel(x, batch, scale, bias):
    psum, psq, pcnt = _make_stats()(x, batch)
    ab = _finalize(psum, psq, pcnt, scale, bias)
    return _make_norm()(x, batch, ab.reshape(-1))


# speedup vs baseline: 1.0120x; 1.0040x over previous
import functools

import jax
import jax.numpy as jnp
from jax import lax
from jax.experimental import pallas as pl
from jax.experimental.pallas import tpu as pltpu
from jax.experimental.pallas import tpu_sc as plsc

N = 50000
D = 256
S = 256
L = 16
NC = 2
NS = 16
NW = NC * NS
BR = 112
NBLK = (N + BR - 1) // BR
KMAX = (NBLK + NW - 1) // NW
H = D // 2
FH = H // L
FV = D // L

_f32 = jnp.float32
_i32 = jnp.int32


RPT = 1568
SBR = 32
SKB = RPT // SBR


def _stats_body(x_hbm, batch_hbm, psum, psq, pcnt,
                idxall, xb0, xb1, asum, asq, acnt, xs0, xs1,
                ws0, ws1, ws2):
    cid = lax.axis_index("c")
    sid = lax.axis_index("s")
    wid = sid * NC + cid
    xb = (xb0, xb1)
    xsem = (xs0, xs1)
    ones = jnp.ones((L,), _f32)
    sixteens = jnp.full((L,), 16.0, _f32)
    iota = lax.iota(_i32, L)

    base = RPT * wid
    tend = jnp.minimum(base + RPT, N)
    astart = pl.multiple_of(jnp.minimum(base, N - RPT), 8)
    pltpu.sync_copy(batch_hbm.at[pl.ds(astart, RPT)], idxall)
    nblk = (tend - base + SBR - 1) // SBR

    def pstart(k):
        return pl.multiple_of(jnp.minimum(base + SBR * k, N - SBR), 8)

    for h in range(2):
        def issue(k, i):
            pltpu.async_copy(
                x_hbm.at[pl.ds(pstart(k), SBR), pl.ds(h * H, H)],
                xb[i], xsem[i])

        for i in range(2):
            @pl.when(i < nblk)
            def _():
                issue(i, i)

        if h == 1:
            pltpu.make_async_copy(asum, psum.at[0, wid], ws0).wait()
            pltpu.make_async_copy(asq, psq.at[0, wid], ws1).wait()
            pltpu.make_async_copy(acnt, pcnt.at[wid], ws2).wait()

        def zero_body(r, _):
            for f in range(FH):
                asum[r, pl.ds(f * L, L)] = jnp.zeros((L,), _f32)
                asq[r, pl.ds(f * L, L)] = jnp.zeros((L,), _f32)
            if h == 0:
                acnt[r, pl.ds(0, L)] = jnp.zeros((L,), _f32)
            return 0
        lax.fori_loop(0, S + 1, zero_body, 0)

        def ring_body(t, _):
            for i in range(2):
                k = 2 * t + i

                @pl.when(k < nblk)
                def _():
                    ps = pstart(k)
                    pltpu.make_async_copy(
                        x_hbm.at[pl.ds(ps, SBR), pl.ds(h * H, H)],
                        xb[i], xsem[i]).wait()
                    vfrom = base + SBR * k
                    vto = jnp.minimum(vfrom + SBR, tend)
                    loffk = ps - astart

                    def group_body(g, _):
                        bv = idxall[pl.ds(loffk + g * L, L)]
                        pos = ps + g * L + iota
                        validv = (pos >= vfrom) & (pos < vto)
                        u = jnp.where(validv, bv, S)
                        u0 = u[0]
                        uniform = (u0 == u[L - 1]) & (u0 < S)

                        @pl.when(uniform)
                        def _():
                            seg = u0
                            for f in range(FH):
                                acs = jnp.zeros((L,), _f32)
                                acq = jnp.zeros((L,), _f32)
                                for j in range(L):
                                    xv = xb[i][g * L + j, pl.ds(f * L, L)]
                                    acs = acs + xv
                                    acq = acq + xv * xv
                                plsc.addupdate(
                                    asum.at[seg, pl.ds(f * L, L)], acs)
                                plsc.addupdate(
                                    asq.at[seg, pl.ds(f * L, L)], acq)
                            if h == 0:
                                plsc.addupdate(
                                    acnt.at[seg, pl.ds(0, L)], sixteens)

                        @pl.when(jnp.logical_not(uniform))
                        def _():
                            for j in range(L):
                                seg = u[j]
                                for f in range(FH):
                                    xv = xb[i][g * L + j, pl.ds(f * L, L)]
                                    plsc.addupdate(
                                        asum.at[seg, pl.ds(f * L, L)], xv)
                                    plsc.addupdate(
                                        asq.at[seg, pl.ds(f * L, L)],
                                        xv * xv)
                                if h == 0:
                                    plsc.addupdate(
                                        acnt.at[seg, pl.ds(0, L)], ones)
                        return 0
                    lax.fori_loop(0, SBR // L, group_body, 0)

                    @pl.when(k + 2 < nblk)
                    def _():
                        issue(k + 2, i)
            return 0

        lax.fori_loop(0, (SKB + 1) // 2, ring_body, 0)

        pltpu.async_copy(asum, psum.at[h, wid], ws0)
        pltpu.async_copy(asq, psq.at[h, wid], ws1)
        if h == 0:
            pltpu.async_copy(acnt, pcnt.at[wid], ws2)

    pltpu.make_async_copy(asum, psum.at[1, wid], ws0).wait()
    pltpu.make_async_copy(asq, psq.at[1, wid], ws1).wait()


@functools.cache
def _make_stats():
  mesh = plsc.VectorSubcoreMesh(
      core_axis_name="c", subcore_axis_name="s",
      num_cores=NC, num_subcores=NS)
  return functools.partial(
    pl.kernel,
    mesh=mesh,
    out_type=(
        jax.ShapeDtypeStruct((2, NW, S + 1, H), _f32),
        jax.ShapeDtypeStruct((2, NW, S + 1, H), _f32),
        jax.ShapeDtypeStruct((NW, S + 1, L), _f32),
    ),
    scratch_types=[
        pltpu.VMEM((RPT,), _i32),
        pltpu.VMEM((SBR, H), _f32),
        pltpu.VMEM((SBR, H), _f32),
        pltpu.VMEM((S + 1, H), _f32),
        pltpu.VMEM((S + 1, H), _f32),
        pltpu.VMEM((S + 1, L), _f32),
        pltpu.SemaphoreType.DMA,
        pltpu.SemaphoreType.DMA,
        pltpu.SemaphoreType.DMA,
        pltpu.SemaphoreType.DMA,
        pltpu.SemaphoreType.DMA,
    ],
  )(_stats_body)


def _finalize_body(psum_ref, psq_ref, pcnt_ref, scale_ref, bias_ref, ab_ref):
    s0 = jnp.sum(psum_ref[0], axis=0)[:S]
    s1 = jnp.sum(psum_ref[1], axis=0)[:S]
    q0 = jnp.sum(psq_ref[0], axis=0)[:S]
    q1 = jnp.sum(psq_ref[1], axis=0)[:S]
    s = jnp.concatenate([s0, s1], axis=1)
    q = jnp.concatenate([q0, q1], axis=1)
    c = jnp.sum(pcnt_ref[...], axis=0)[:S, 0:1]
    c_safe = jnp.maximum(c, 1.0)
    mean = s / c_safe
    denom = jnp.maximum(c - 1.0, 1.0)
    var = jnp.maximum((q - c * mean * mean) / denom, 0.0)
    std = jnp.sqrt(var)
    a = scale_ref[...][None, :] / (std + 1e-5)
    bb = bias_ref[...][None, :] - mean * a
    ab_ref[0:S, 0:D] = a
    ab_ref[0:S, D:2 * D] = bb
    ab_ref[S:, :] = jnp.zeros((16, 2 * D), _f32)


def _finalize(psum, psq, pcnt, scale, bias):
    return pl.pallas_call(
        _finalize_body,
        out_shape=jax.ShapeDtypeStruct((S + 16, 2 * D), _f32),
    )(psum, psq, pcnt, scale, bias)


NRPT = 1600
NB = 80
NKB = (NRPT + NB - 1) // NB
W = 8


def _norm_body(x_hbm, batch_hbm, ab_hbm, out_hbm,
               ib0, ib1, xb0, xb1, ob0, ob1, aw0, aw1, tmp,
               is0, is1, xs0, xs1, as0, as1, os0, os1):
    cid = lax.axis_index("c")
    sid = lax.axis_index("s")
    wid = sid * NC + cid
    ib = (ib0, ib1)
    xb = (xb0, xb1)
    ob = (ob0, ob1)
    aw = (aw0, aw1)
    isem = (is0, is1)
    xsem = (xs0, xs1)
    asem = (as0, as1)
    osem = (os0, os1)

    base = NRPT * wid
    tend = jnp.minimum(base + NRPT, N)
    nblk = (tend - base + NB - 1) // NB

    def pstart(k):
        return pl.multiple_of(jnp.minimum(base + NB * k, N - NB), 8)

    def u0last(i):
        v0 = ib[i][pl.ds(0, L)]
        v2 = ib[i][pl.ds(NB - L, L)]
        return v0[0], v2[L - 1]

    def issue_ix(k, i):
        ps = pstart(k)
        pltpu.async_copy(batch_hbm.at[pl.ds(ps, NB)], ib[i], isem[i])
        pltpu.async_copy(x_hbm.at[pl.ds(ps, NB)], xb[i], xsem[i])

    def issue_ab(i):
        u0, ul = u0last(i)

        @pl.when(ul - u0 < W)
        def _():
            off = pl.multiple_of(u0 * 2 * D, 8)
            pltpu.async_copy(ab_hbm.at[pl.ds(off, W * 2 * D)], aw[i], asem[i])

    def wait_ab(i):
        u0, ul = u0last(i)

        @pl.when(ul - u0 < W)
        def _():
            off = pl.multiple_of(u0 * 2 * D, 8)
            pltpu.make_async_copy(
                ab_hbm.at[pl.ds(off, W * 2 * D)], aw[i], asem[i]).wait()

    issue_ix(0, 0)
    pltpu.make_async_copy(
        batch_hbm.at[pl.ds(pstart(0), NB)], ib[0], isem[0]).wait()
    issue_ab(0)

    @pl.when(1 < nblk)
    def _():
        issue_ix(1, 1)

    def pair_body(t, _):
        for i in range(2):
            k = 2 * t + i
            inext = (i + 1) & 1

            @pl.when(k < nblk)
            def _():
                ps = pstart(k)

                @pl.when(k + 1 < nblk)
                def _():
                    pltpu.make_async_copy(
                        batch_hbm.at[pl.ds(pstart(k + 1), NB)],
                        ib[inext], isem[inext]).wait()
                    issue_ab(inext)

                pltpu.make_async_copy(
                    x_hbm.at[pl.ds(ps, NB)], xb[i], xsem[i]).wait()
                wait_ab(i)

                @pl.when(k >= 2)
                def _():
                    pltpu.make_async_copy(
                        ob[i], out_hbm.at[pl.ds(pstart(k - 2), NB)],
                        osem[i]).wait()

                u0, ul = u0last(i)

                @pl.when(ul - u0 < W)
                def _():
                    def group_body(g, _):
                        idv = ib[i][pl.ds(g * L, L)]
                        guni = idv[0] == idv[L - 1]

                        @pl.when(guni)
                        def _():
                            rbase = (idv[0] - u0) * 2 * D

                            def cbody(c, _):
                                for ff in range(4):
                                    off = c * 4 * L + ff * L
                                    av = aw[i][pl.ds(rbase + off, L)]
                                    bv = aw[i][pl.ds(rbase + D + off, L)]
                                    for j in range(L):
                                        r = g * L + j
                                        ob[i][r, pl.ds(off, L)] = (
                                            xb[i][r, pl.ds(off, L)] * av
                                            + bv)
                                return 0
                            lax.fori_loop(0, FV // 4, cbody, 0)

                        @pl.when(jnp.logical_not(guni))
                        def _():
                            for j in range(L):
                                r = g * L + j
                                rbase = (idv[j] - u0) * 2 * D

                                def fbody(f, _):
                                    off = f * L
                                    av = aw[i][pl.ds(rbase + off, L)]
                                    bv = aw[i][pl.ds(rbase + D + off, L)]
                                    ob[i][r, pl.ds(off, L)] = (
                                        xb[i][r, pl.ds(off, L)] * av + bv)
                                    return 0
                                lax.fori_loop(0, FV, fbody, 0)
                        return 0
                    lax.fori_loop(0, NB // L, group_body, 0)

                @pl.when(ul - u0 >= W)
                def _():
                    def sgroup_body(g, _):
                        idv = ib[i][pl.ds(g * L, L)]
                        for j in range(L):
                            r = g * L + j
                            soff = pl.multiple_of(idv[j] * 2 * D, 8)
                            pltpu.sync_copy(
                                ab_hbm.at[pl.ds(soff, 2 * D)], tmp)

                            def fbody(f, _):
                                off = f * L
                                av = tmp[pl.ds(off, L)]
                                bv = tmp[pl.ds(D + off, L)]
                                ob[i][r, pl.ds(off, L)] = (
                                    xb[i][r, pl.ds(off, L)] * av + bv)
                                return 0
                            lax.fori_loop(0, FV, fbody, 0)
                        return 0
                    lax.fori_loop(0, NB // L, sgroup_body, 0)

                pltpu.async_copy(ob[i], out_hbm.at[pl.ds(ps, NB)], osem[i])

                @pl.when(k + 2 < nblk)
                def _():
                    issue_ix(k + 2, i)
        return 0

    lax.fori_loop(0, (NKB + 1) // 2, pair_body, 0)

    for i in range(2):
        @pl.when(i < nblk)
        def _():
            pltpu.make_async_copy(
                ob[i], out_hbm.at[pl.ds(pstart(0), NB)], osem[i]).wait()


@functools.cache
def _make_norm():
  mesh = plsc.VectorSubcoreMesh(
      core_axis_name="c", subcore_axis_name="s",
      num_cores=NC, num_subcores=NS)
  return functools.partial(
    pl.kernel,
    mesh=mesh,
    out_type=jax.ShapeDtypeStruct((N, D), _f32),
    scratch_types=[
        pltpu.VMEM((NB,), _i32),
        pltpu.VMEM((NB,), _i32),
        pltpu.VMEM((NB, D), _f32),
        pltpu.VMEM((NB, D), _f32),
        pltpu.VMEM((NB, D), _f32),
        pltpu.VMEM((NB, D), _f32),
        pltpu.VMEM((W * 2 * D,), _f32),
        pltpu.VMEM((W * 2 * D,), _f32),
        pltpu.VMEM((2 * D,), _f32),
        pltpu.SemaphoreType.DMA,
        pltpu.SemaphoreType.DMA,
        pltpu.SemaphoreType.DMA,
        pltpu.SemaphoreType.DMA,
        pltpu.SemaphoreType.DMA,
        pltpu.SemaphoreType.DMA,
        pltpu.SemaphoreType.DMA,
        pltpu.SemaphoreType.DMA,
    ],
  )(_norm_body)


def kernel(x, batch, scale, bias):
    psum, psq, pcnt = _make_stats()(x, batch)
    ab = _finalize(psum, psq, pcnt, scale, bias)
    return _make_norm()(x, batch, ab.reshape(-1))
